# Initial kernel scaffold; baseline (speedup 1.0000x reference)
import jax
import jax.numpy as jnp
from jax.experimental import pallas as pl


def kernel(labels, centers):
    raise NotImplementedError("write your pallas kernel here")


# speedup vs baseline: 1.5490x; 1.5490x over previous
import functools

import jax
import jax.numpy as jnp
from jax import lax
from jax.experimental import pallas as pl
from jax.experimental.pallas import tpu as pltpu
from jax.experimental.pallas import tpu_sc as plsc

NUM_CLASSES = 100000
FEATURE_DIM = 128
BATCH = 16384

_NC = 2
_NS = 16
_NW = _NC * _NS
_CHUNK = 128
_NCHUNK = BATCH // (_NW * _CHUNK)


def _gather_kernel(centers_hbm, idx_hbm, out_hbm, idx_v, rows_v, sems):
    wid = lax.axis_index("s") * _NC + lax.axis_index("c")
    base = wid * _NCHUNK
    pltpu.sync_copy(idx_hbm.at[pl.ds(base, _NCHUNK)], idx_v)
    copies = []
    for j in range(_NCHUNK):
        copies.append(
            pltpu.async_copy(centers_hbm.at[idx_v.at[j]], rows_v.at[j], sems.at[j])
        )
    for c in copies:
        c.wait()
    pltpu.sync_copy(rows_v, out_hbm.at[pl.ds(base, _NCHUNK)])


@jax.jit
def kernel(labels, centers):
    idx2d = labels.astype(jnp.int32).reshape(_NW * _NCHUNK, _CHUNK)
    mesh = plsc.VectorSubcoreMesh(core_axis_name="c", subcore_axis_name="s")
    out3d = pl.kernel(
        _gather_kernel,
        mesh=mesh,
        out_type=jax.ShapeDtypeStruct((_NW * _NCHUNK, _CHUNK, FEATURE_DIM), jnp.float32),
        scratch_types=[
            pltpu.VMEM((_NCHUNK, _CHUNK), jnp.int32),
            pltpu.VMEM((_NCHUNK, _CHUNK, FEATURE_DIM), jnp.float32),
            pltpu.SemaphoreType.DMA((_NCHUNK,)),
        ],
    )(centers, idx2d)
    return out3d.reshape(BATCH, FEATURE_DIM)
